# Initial kernel scaffold; baseline (speedup 1.0000x reference)
import jax
import jax.numpy as jnp
from jax.experimental import pallas as pl


def kernel(x, edge_index, B, Pi):
    raise NotImplementedError("write your pallas kernel here")


# speedup vs baseline: 5.8990x; 5.8990x over previous
import functools

import jax
import jax.numpy as jnp
from jax import lax
from jax.experimental import pallas as pl
from jax.experimental.pallas import tpu as pltpu
from jax.experimental.pallas import tpu_sc as plsc

N_NODES = 100000
C = 20
M = 128
N_GEN = 16

_NC = 2
_NS = 16
_NW = _NC * _NS
_N_PAD = 102400
_B_PER_W = _N_PAD // _NW


def _table_body(b_ref, pi_ref, t_ref):
    b = b_ref[...]
    pi = pi_ref[...]
    sm_b = jax.nn.softmax(b, axis=1)
    sm_pi = jax.nn.softmax(pi, axis=0)
    acc = jnp.sum(sm_pi[:, None, :] * sm_b, axis=0)
    t_ref[...] = jnp.log(acc + C * 1e-12)


def _compute_table(B, Pi):
    return pl.pallas_call(
        _table_body,
        out_shape=jax.ShapeDtypeStruct((M, N_GEN), jnp.float32),
    )(B, Pi)


_MESH = plsc.VectorSubcoreMesh(core_axis_name="c", subcore_axis_name="s")


@functools.partial(
    pl.kernel,
    mesh=_MESH,
    out_type=jax.ShapeDtypeStruct((_N_PAD, N_GEN), jnp.float32),
    scratch_types=[
        pltpu.VMEM((_B_PER_W,), jnp.int32),
        pltpu.VMEM((_B_PER_W, N_GEN), jnp.float32),
        pltpu.SemaphoreType.DMA,
    ],
    compiler_params=pltpu.CompilerParams(use_tc_tiling_on_sc=False),
)
def _gather_kernel(idx_hbm, table_hbm, out_hbm, idx_v, rows_v, sem):
    wid = lax.axis_index("s") * _NC + lax.axis_index("c")
    base = wid * _B_PER_W
    pltpu.sync_copy(idx_hbm.at[pl.ds(base, _B_PER_W)], idx_v)
    pltpu.async_copy(table_hbm.at[idx_v], rows_v, sem).wait()
    pltpu.sync_copy(rows_v, out_hbm.at[pl.ds(base, _B_PER_W)])


def kernel(x, edge_index, B, Pi):
    del edge_index
    table = _compute_table(B, Pi)
    x_pad = jnp.concatenate(
        [x, jnp.zeros((_N_PAD - N_NODES,), dtype=jnp.int32)])
    rows = _gather_kernel(x_pad, table)
    return rows[:N_NODES, None, :]


# speedup vs baseline: 8.7814x; 1.4886x over previous
import functools

import jax
import jax.numpy as jnp
from jax import lax
from jax.experimental import pallas as pl
from jax.experimental.pallas import tpu as pltpu
from jax.experimental.pallas import tpu_sc as plsc

N_NODES = 100000
C = 20
M = 128
N_GEN = 16

_NC = 2
_NS = 16
_NW = _NC * _NS
_B_PER_W = 3128
_B_LAST = N_NODES - (_NW - 1) * _B_PER_W


def _table_body(b_ref, pi_ref, t_ref):
    b = b_ref[...]
    pi = pi_ref[...]
    sm_b = jax.nn.softmax(b, axis=1)
    sm_pi = jax.nn.softmax(pi, axis=0)
    acc = jnp.sum(sm_pi[:, None, :] * sm_b, axis=0)
    t_ref[...] = jnp.log(acc + C * 1e-12)


def _compute_table(B, Pi):
    return pl.pallas_call(
        _table_body,
        out_shape=jax.ShapeDtypeStruct((M, N_GEN), jnp.float32),
    )(B, Pi)


_MESH = plsc.VectorSubcoreMesh(core_axis_name="c", subcore_axis_name="s")


@functools.partial(
    pl.kernel,
    mesh=_MESH,
    out_type=jax.ShapeDtypeStruct((N_NODES, N_GEN), jnp.float32),
    scratch_types=[
        pltpu.VMEM((_B_PER_W,), jnp.int32),
        pltpu.VMEM((_B_PER_W, N_GEN), jnp.float32),
        pltpu.SemaphoreType.DMA,
    ],
    compiler_params=pltpu.CompilerParams(use_tc_tiling_on_sc=False),
)
def _gather_kernel(idx_hbm, table_hbm, out_hbm, idx_v, rows_v, sem):
    wid = lax.axis_index("s") * _NC + lax.axis_index("c")
    base = wid * _B_PER_W

    @pl.when(wid < _NW - 1)
    def _full_chunk():
        pltpu.sync_copy(idx_hbm.at[pl.ds(base, _B_PER_W)], idx_v)
        pltpu.async_copy(table_hbm.at[idx_v], rows_v, sem).wait()
        pltpu.sync_copy(rows_v, out_hbm.at[pl.ds(base, _B_PER_W)])

    @pl.when(wid == _NW - 1)
    def _tail_chunk():
        pltpu.sync_copy(idx_hbm.at[pl.ds(base, _B_LAST)],
                        idx_v.at[pl.ds(0, _B_LAST)])
        pltpu.async_copy(table_hbm.at[idx_v.at[pl.ds(0, _B_LAST)]],
                         rows_v.at[pl.ds(0, _B_LAST)], sem).wait()
        pltpu.sync_copy(rows_v.at[pl.ds(0, _B_LAST)],
                        out_hbm.at[pl.ds(base, _B_LAST)])


def kernel(x, edge_index, B, Pi):
    del edge_index
    table = _compute_table(B, Pi)
    rows = _gather_kernel(x, table)
    return rows[:, None, :]
